# Initial kernel scaffold; baseline (speedup 1.0000x reference)
import jax
import jax.numpy as jnp
from jax.experimental import pallas as pl


def kernel(x, mapping, table):
    raise NotImplementedError("write your pallas kernel here")


# speedup vs baseline: 4.0259x; 4.0259x over previous
import functools

import jax
import jax.numpy as jnp
from jax import lax
from jax.experimental import pallas as pl
from jax.experimental.pallas import tpu as pltpu
from jax.experimental.pallas import tpu_sc as plsc

_B = 1024
_I = 2048
_O = 2048
_NI = 4
_NC = 16
_L = 16
_NCORES = 2
_NSUB = 16
_NW = _NCORES * _NSUB
_BCHUNK = 16
_NPASS = _B // (_NW * _BCHUNK)
_NJG = _O // _L


def _sc_body(x_hbm, map_hbm, tab_hbm, out_hbm, x_v, map_v, tab_v, out_v):
    cid = lax.axis_index("c")
    sid = lax.axis_index("s")
    wid = sid * _NCORES + cid

    pltpu.sync_copy(map_hbm, map_v)
    pltpu.sync_copy(tab_hbm, tab_v)

    for p in range(_NPASS):
        b0 = (wid * _NPASS + p) * _BCHUNK
        pltpu.sync_copy(x_hbm.at[pl.ds(b0 * _I, _BCHUNK * _I)], x_v)

        def jg_body(jg, carry):
            base = jg * _L
            idx = [map_v[pl.ds(jg * (_NI * _L) + k * _L, _L)] for k in range(_NI)]
            t = [tab_v[pl.ds(c * _O + base, _L)] for c in range(_NC)]
            A = t[:4]
            Bc = [t[c + 4] - t[c] for c in range(4)]
            Cc = [t[c + 8] - t[c] for c in range(4)]
            Dc = [(t[c + 12] - t[c + 8]) - Bc[c] for c in range(4)]
            for b in range(_BCHUNK):
                m = [plsc.load_gather(x_v, [idx[k] + (b * _I)])
                     for k in range(_NI)]
                m23 = m[2] * m[3]
                v = [A[c] + Bc[c] * m[2] + Cc[c] * m[3] + Dc[c] * m23
                     for c in range(4)]
                u0 = v[0] + (v[2] - v[0]) * m[1]
                u1 = v[1] + (v[3] - v[1]) * m[1]
                out_v[pl.ds(b * _O + base, _L)] = u0 + (u1 - u0) * m[0]
            return carry

        lax.fori_loop(0, _NJG, jg_body, 0)
        pltpu.sync_copy(out_v, out_hbm.at[pl.ds(b0 * _O, _BCHUNK * _O)])


_lut_sc = functools.partial(
    pl.kernel,
    mesh=plsc.VectorSubcoreMesh(core_axis_name="c", subcore_axis_name="s"),
    compiler_params=pltpu.CompilerParams(needs_layout_passes=False),
    out_type=jax.ShapeDtypeStruct((_B * _O,), jnp.float32),
    scratch_types=[
        pltpu.VMEM((_BCHUNK * _I,), jnp.float32),
        pltpu.VMEM((_NI * _O,), jnp.int32),
        pltpu.VMEM((_NC * _O,), jnp.float32),
        pltpu.VMEM((_BCHUNK * _O,), jnp.float32),
    ],
)(_sc_body)


def kernel(x, mapping, table):
    map_r = mapping.reshape(_NJG, _L, _NI).transpose(0, 2, 1).reshape(-1)
    tab_r = table.T.reshape(-1)
    return _lut_sc(x.reshape(-1), map_r, tab_r).reshape(_B, _O)


# speedup vs baseline: 4.9160x; 1.2211x over previous
import functools

import jax
import jax.numpy as jnp
from jax import lax
from jax.experimental import pallas as pl
from jax.experimental.pallas import tpu as pltpu
from jax.experimental.pallas import tpu_sc as plsc

_B = 1024
_I = 2048
_O = 2048
_NI = 4
_NC = 16
_L = 16
_NCORES = 2
_NSUB = 16
_NW = _NCORES * _NSUB
_BCHUNK = 16
_NPASS = _B // (_NW * _BCHUNK)
_NJG = _O // _L


def _sc_body(x_hbm, map_hbm, tab_hbm, out_hbm, x_v, map_v, tab_v, out_v):
    cid = lax.axis_index("c")
    sid = lax.axis_index("s")
    wid = sid * _NCORES + cid

    pltpu.sync_copy(map_hbm, map_v)
    pltpu.sync_copy(tab_hbm, tab_v)

    for p in range(_NPASS):
        b0 = (wid * _NPASS + p) * _BCHUNK
        pltpu.sync_copy(x_hbm.at[pl.ds(b0 * _I, _BCHUNK * _I)], x_v)

        def jg_body(jg, carry):
            base = jg * _L
            idx = [map_v[pl.ds(jg * (_NI * _L) + k * _L, _L)] for k in range(_NI)]
            t = [tab_v[pl.ds(c * _O + base, _L)] for c in range(_NC)]
            a = list(t)
            for k in range(_NI):
                bit = 1 << k
                a = [a[s] if not s & bit else a[s] - a[s ^ bit]
                     for s in range(_NC)]
            for b in range(_BCHUNK):
                xb = x_v.at[pl.ds(b * _I, _I)]
                m = [plsc.load_gather(xb, [idx[k]]) for k in range(_NI)]
                f = a
                for k in reversed(range(_NI)):
                    half = 1 << k
                    f = [f[s] + m[k] * f[s + half] for s in range(half)]
                out_v[pl.ds(b * _O + base, _L)] = f[0]
            return carry

        lax.fori_loop(0, _NJG, jg_body, 0)
        pltpu.sync_copy(out_v, out_hbm.at[pl.ds(b0 * _O, _BCHUNK * _O)])


_lut_sc = functools.partial(
    pl.kernel,
    mesh=plsc.VectorSubcoreMesh(core_axis_name="c", subcore_axis_name="s"),
    compiler_params=pltpu.CompilerParams(needs_layout_passes=False),
    out_type=jax.ShapeDtypeStruct((_B * _O,), jnp.float32),
    scratch_types=[
        pltpu.VMEM((_BCHUNK * _I,), jnp.float32),
        pltpu.VMEM((_NI * _O,), jnp.int32),
        pltpu.VMEM((_NC * _O,), jnp.float32),
        pltpu.VMEM((_BCHUNK * _O,), jnp.float32),
    ],
)(_sc_body)


def kernel(x, mapping, table):
    map_r = mapping.reshape(_NJG, _L, _NI).transpose(0, 2, 1).reshape(-1)
    tab_r = table.T.reshape(-1)
    return _lut_sc(x.reshape(-1), map_r, tab_r).reshape(_B, _O)


# speedup vs baseline: 6.6361x; 1.3499x over previous
import functools

import numpy as np

import jax
import jax.numpy as jnp
from jax import lax
from jax.experimental import pallas as pl
from jax.experimental.pallas import tpu as pltpu
from jax.experimental.pallas import tpu_sc as plsc

_B = 1024
_I = 2048
_O = 2048
_NI = 4
_NC = 16
_L = 16
_NCORES = 2
_NSUB = 16
_NW = _NCORES * _NSUB
_BCHUNK = 16
_NPASS = _B // (_NW * _BCHUNK)
_NJG = _O // _L


def _moebius_matrix() -> np.ndarray:
    m = np.zeros((_NC, _NC), dtype=np.float32)
    for s in range(_NC):
        for c in range(_NC):
            if c & s == c:
                m[c, s] = (-1.0) ** bin(s ^ c).count("1")
    return m


_MOEBIUS = _moebius_matrix()


def _sc_body(x_hbm, map_hbm, tab_hbm, out_hbm, x_v, map_v, tab_v, out_v, sem):
    cid = lax.axis_index("c")
    sid = lax.axis_index("s")
    wid = sid * _NCORES + cid

    b0 = wid * _NPASS * _BCHUNK
    c1 = pltpu.async_copy(map_hbm, map_v, sem)
    c2 = pltpu.async_copy(tab_hbm, tab_v, sem)
    c3 = pltpu.async_copy(x_hbm.at[pl.ds(b0, _BCHUNK), :], x_v, sem)
    c1.wait()
    c2.wait()
    c3.wait()

    for p in range(_NPASS):
        b0 = (wid * _NPASS + p) * _BCHUNK

        @plsc.parallel_loop(0, _NJG)
        def jg_body(jg):
            base = jg * _L
            idx = [map_v[pl.ds(jg * (_NI * _L) + k * _L, _L)]
                   for k in reversed(range(_NI))]
            idx.reverse()
            a = [tab_v[pl.ds(s * _O + base, _L)] for s in range(_NC)]
            for b in range(_BCHUNK):
                bs = jnp.full((_L,), b, jnp.int32)
                m = [plsc.load_gather(x_v, [bs, idx[k]])
                     for k in reversed(range(_NI))]
                m.reverse()
                f = a
                for k in reversed(range(_NI)):
                    half = 1 << k
                    f = [f[s] + m[k] * f[s + half] for s in range(half)]
                out_v[b, pl.ds(base, _L)] = f[0]

        if p + 1 < _NPASS:
            co = pltpu.async_copy(out_v, out_hbm.at[pl.ds(b0, _BCHUNK), :], sem)
            b1 = b0 + _BCHUNK
            cx = pltpu.async_copy(x_hbm.at[pl.ds(b1, _BCHUNK), :], x_v, sem)
            co.wait()
            cx.wait()
        else:
            pltpu.sync_copy(out_v, out_hbm.at[pl.ds(b0, _BCHUNK), :])


_lut_sc = functools.partial(
    pl.kernel,
    mesh=plsc.VectorSubcoreMesh(core_axis_name="c", subcore_axis_name="s"),
    compiler_params=pltpu.CompilerParams(needs_layout_passes=False),
    out_type=jax.ShapeDtypeStruct((_B, _O), jnp.float32),
    scratch_types=[
        pltpu.VMEM((_BCHUNK, _I), jnp.float32),
        pltpu.VMEM((_NI * _O,), jnp.int32),
        pltpu.VMEM((_NC * _O,), jnp.float32),
        pltpu.VMEM((_BCHUNK, _O), jnp.float32),
        pltpu.SemaphoreType.DMA,
    ],
)(_sc_body)


def kernel(x, mapping, table):
    map_r = mapping.reshape(_NJG, _L, _NI).transpose(0, 2, 1).reshape(-1)
    tab_r = (table @ jnp.asarray(_MOEBIUS)).T.reshape(-1)
    return _lut_sc(x, map_r, tab_r)
